# baseline (speedup 1.0000x reference)
import functools

import jax
import jax.numpy as jnp
from jax import lax
from jax.experimental import pallas as pl
from jax.experimental.pallas import tpu as pltpu
from jax.experimental.pallas import tpu_sc as plsc

N = 10000
E = 320000
F = 128

C = 128
HALVES = 2
NCH = 80
NCHUNK = HALVES * NCH
TILES = 16
EPAD = TILES * NCHUNK * C
OUTN = 10240
NACC = OUTN
ZROWS = NACC // TILES
ORS = OUTN // TILES

BR = 1000
NB = N // BR


def _build_table_body(h_ref, fd_ref, wr_ref, ur_ref, br_ref, out_ref):
    i = pl.program_id(0)

    @pl.when(i < NB)
    def _copy():
        out_ref[...] = h_ref[...]

    @pl.when(i >= NB)
    def _compute():
        hv = h_ref[...]
        r = jax.nn.sigmoid(
            jnp.dot(fd_ref[...], wr_ref[...], preferred_element_type=jnp.float32)
            + jnp.dot(hv, ur_ref[...], preferred_element_type=jnp.float32)
            + br_ref[...]
        )
        out_ref[...] = r * hv


def _build_table(h, f_dst, wr, ur, br):
    return pl.pallas_call(
        _build_table_body,
        grid=(2 * NB,),
        in_specs=[
            pl.BlockSpec((BR, F), lambda i: (jnp.where(i < NB, i, i - NB), 0)),
            pl.BlockSpec((BR, F), lambda i: (jnp.where(i < NB, 0, i - NB), 0)),
            pl.BlockSpec((F, F), lambda i: (0, 0)),
            pl.BlockSpec((F, F), lambda i: (0, 0)),
            pl.BlockSpec((1, F), lambda i: (0, 0)),
        ],
        out_specs=pl.BlockSpec((BR, F), lambda i: (i, 0)),
        out_shape=jax.ShapeDtypeStruct((2 * N, F), jnp.float32),
    )(h, f_dst, wr, ur, br)


@functools.cache
def _make_segment_sums():
    mesh = plsc.VectorSubcoreMesh(core_axis_name="c", subcore_axis_name="s")

    @functools.partial(
        pl.kernel,
        out_type=jax.ShapeDtypeStruct((2, OUTN, F), jnp.float32),
        mesh=mesh,
        compiler_params=pltpu.CompilerParams(use_tc_tiling_on_sc=False),
        scratch_types=[
            pltpu.VMEM((NCH, C), jnp.int32),
            pltpu.VMEM((NCH, C), jnp.int32),
            pltpu.VMEM((C, F), jnp.float32),
            pltpu.VMEM_SHARED((NACC, F), jnp.float32),
            pltpu.SemaphoreType.DMA,
        ],
    )
    def seg(t_hbm, src_hbm, dst_hbm, zeros_hbm, out_hbm,
            src_v, dst_v, rows_v, acc_sh, sem):
        c = lax.axis_index("c")
        s = lax.axis_index("s")
        pltpu.sync_copy(zeros_hbm, acc_sh.at[pl.ds(s * ZROWS, ZROWS)])
        plsc.subcore_barrier()

        for half in range(HALVES):
            pltpu.sync_copy(src_hbm.at[c, s, half], src_v)
            pltpu.sync_copy(dst_hbm.at[s, half], dst_v)

            def body(j, carry):
                pltpu.async_copy(t_hbm.at[src_v.at[j]], rows_v, sem).wait()
                pltpu.sync_copy(rows_v, acc_sh.at[dst_v.at[j]], add=True)
                return carry

            lax.fori_loop(0, NCH, body, 0)

        plsc.subcore_barrier()
        pltpu.sync_copy(acc_sh.at[pl.ds(s * ORS, ORS)],
                        out_hbm.at[c, pl.ds(s * ORS, ORS)])

    return seg


def _segment_sums(table, src2, dst_r, zeros):
    return _make_segment_sums()(table, src2, dst_r, zeros)


def _gate_body(fs_ref, s_ref, srh_ref, wz_ref, uz_ref, bz_ref,
               w_ref, u_ref, b_ref, out_ref):
    fs = fs_ref[...]
    sv = s_ref[0]
    srh = srh_ref[0]
    z = jax.nn.sigmoid(
        jnp.dot(fs, wz_ref[...], preferred_element_type=jnp.float32)
        + jnp.dot(sv, uz_ref[...], preferred_element_type=jnp.float32)
        + bz_ref[...]
    )
    ht = jnp.tanh(
        jnp.dot(fs, w_ref[...], preferred_element_type=jnp.float32)
        + jnp.dot(srh, u_ref[...], preferred_element_type=jnp.float32)
        + b_ref[...]
    )
    out_ref[...] = (1.0 - z) * sv + z * ht


def _gate(f_src, seg, wz, uz, bz, w, u, b):
    full = lambda i: (0, 0)
    return pl.pallas_call(
        _gate_body,
        grid=(NB,),
        in_specs=[
            pl.BlockSpec((BR, F), lambda i: (i, 0)),
            pl.BlockSpec((1, BR, F), lambda i: (0, i, 0)),
            pl.BlockSpec((1, BR, F), lambda i: (1, i, 0)),
            pl.BlockSpec((F, F), full),
            pl.BlockSpec((F, F), full),
            pl.BlockSpec((1, F), full),
            pl.BlockSpec((F, F), full),
            pl.BlockSpec((F, F), full),
            pl.BlockSpec((1, F), full),
        ],
        out_specs=pl.BlockSpec((BR, F), lambda i: (i, 0)),
        out_shape=jax.ShapeDtypeStruct((N, F), jnp.float32),
    )(f_src, seg, seg, wz, uz, bz, w, u, b)


def kernel(h, f_src, f_dst, edge_index, wz, uz, bz, wr, ur, br, w, u, b):
    src = edge_index[0]
    dst = edge_index[1]
    pad = EPAD - E
    src_p = jnp.concatenate([src, jnp.zeros((pad,), jnp.int32)])
    dst_p = jnp.concatenate(
        [dst, N + (jnp.arange(pad, dtype=jnp.int32) % (OUTN - N))])
    src_r = src_p.reshape(TILES, HALVES, NCH, C)
    src2 = jnp.stack([src_r, src_r + N])
    dst_r = dst_p.reshape(TILES, HALVES, NCH, C)
    zeros = jnp.zeros((ZROWS, F), jnp.float32)

    table = _build_table(h, f_dst, wr, ur, br)
    seg = _segment_sums(table, src2, dst_r, zeros)
    return _gate(f_src, seg, wz, uz, bz, w, u, b)
